# Initial kernel scaffold; baseline (speedup 1.0000x reference)
import jax
import jax.numpy as jnp
from jax.experimental import pallas as pl


def kernel(x_nchw, weight, bias):
    raise NotImplementedError("write your pallas kernel here")


# speedup vs baseline: 12.2693x; 12.2693x over previous
import functools

import jax
import jax.numpy as jnp
from jax.experimental import pallas as pl
from jax.experimental.pallas import tpu as pltpu

KK = 25
PAD = 12
HOFF = 16
RT = 16
CT = 8
NB = 256
WBLK = 16
SW = CT + 2 * PAD
SH = 48
KDIM = SW * SH


def _round_up(x, m):
    return ((x + m - 1) // m) * m


def _banded_a(weight):
    w2 = weight.reshape(KK, KK).astype(jnp.float32)
    dxs = jnp.arange(KK)
    cs = jnp.arange(CT)
    wls = jnp.arange(SW)
    xm = (wls[None, None, :] == cs[None, :, None] + dxs[:, None, None])
    dys = jnp.arange(KK)
    rs = jnp.arange(RT)
    hls = jnp.arange(SH)
    ym = (hls[None, None, :] == rs[None, :, None] + dys[:, None, None] + 4)
    t1 = jnp.einsum('yx,xcw->ycw', w2, xm.astype(jnp.float32))
    a4 = jnp.einsum('ycw,yrh->crwh', t1, ym.astype(jnp.float32))
    return a4.reshape(CT * RT, KDIM).astype(jnp.bfloat16)


def _conv_sig_kernel(a_ref, b_ref, x_ref, o_ref, xp_ref):
    w_id = pl.program_id(1)
    W, H, _ = x_ref.shape
    Wp, Hp, _ = xp_ref.shape

    @pl.when(w_id == 0)
    def _build():
        xp_ref[:PAD, :, :] = jnp.zeros((PAD, Hp, NB), jnp.bfloat16)
        xp_ref[PAD + W:, :, :] = jnp.zeros((Wp - PAD - W, Hp, NB), jnp.bfloat16)
        xp_ref[PAD:PAD + W, :HOFF, :] = jnp.zeros((W, HOFF, NB), jnp.bfloat16)
        xp_ref[PAD:PAD + W, HOFF + H:, :] = jnp.zeros(
            (W, Hp - HOFF - H, NB), jnp.bfloat16)
        xp_ref[PAD:PAD + W, HOFF:HOFF + H, :] = x_ref[...].astype(jnp.bfloat16)

    bias = b_ref[0]
    a = a_ref[...]
    for wt in range(WBLK // CT):
        wstart = w_id * WBLK + wt * CT
        for ht in range(H // RT):
            hb = ht * RT
            slab = xp_ref[pl.ds(wstart, SW), hb:hb + SH, :].reshape(KDIM, NB)
            acc = jnp.dot(a, slab, preferred_element_type=jnp.float32)
            o_ref[wt * CT:(wt + 1) * CT, hb:hb + RT, :] = (
                jax.nn.sigmoid(acc + bias).reshape(CT, RT, NB))


def _forward(x_nchw, weight, bias):
    N, C, H, W = x_nchw.shape
    assert C == 1
    Wp = _round_up(PAD + W + PAD, 8)
    Hp = _round_up(HOFF + H + PAD + 4, 8)

    a_mat = _banded_a(weight)

    x = jnp.transpose(x_nchw[:, 0, :, :], (2, 1, 0))
    Np = _round_up(N, NB)
    if Np != N:
        x = jnp.pad(x, ((0, 0), (0, 0), (0, Np - N)))

    out = pl.pallas_call(
        _conv_sig_kernel,
        out_shape=jax.ShapeDtypeStruct((W, H, Np), x_nchw.dtype),
        grid=(Np // NB, W // WBLK),
        in_specs=[
            pl.BlockSpec((CT * RT, KDIM), lambda b, w: (0, 0)),
            pl.BlockSpec(memory_space=pltpu.MemorySpace.SMEM),
            pl.BlockSpec((W, H, NB), lambda b, w: (0, 0, b)),
        ],
        out_specs=pl.BlockSpec((WBLK, H, NB), lambda b, w: (w, 0, b)),
        scratch_shapes=[pltpu.VMEM((Wp, Hp, NB), jnp.bfloat16)],
        compiler_params=pltpu.CompilerParams(
            dimension_semantics=("parallel", "arbitrary")),
    )(a_mat, bias.astype(jnp.float32), x)

    out = out[:, :, :N]
    return jnp.transpose(out, (2, 1, 0))[:, None, :, :]


def kernel(x_nchw, weight, bias):
    return _forward(x_nchw, weight, bias)
